# Initial kernel scaffold; baseline (speedup 1.0000x reference)
import jax
import jax.numpy as jnp
from jax.experimental import pallas as pl


def kernel(x, edge_index, batch, W_emb, b_emb, W0, b0, W1, b1, W2, b2, W_act, b_act, W_tgt, b_tgt, W_atom, b_atom):
    raise NotImplementedError("write your pallas kernel here")


# speedup vs baseline: 7.3499x; 7.3499x over previous
import functools

import jax
import jax.numpy as jnp
from jax import lax
from jax.experimental import pallas as pl
from jax.experimental.pallas import tpu as pltpu
from jax.experimental.pallas import tpu_sc as plsc

NN = 10000
EE = 320000
HH = 128
GG = 64
NC = 2
NS = 16
NW = NC * NS
CH = 64
CPT = 160
SEG = 16
NSEG = CPT // SEG
EPT = CPT * CH
E_PAD = NW * EPT
N_ACC = 10240
RPT = N_ACC // NS
DW = 16

_mesh = lambda: plsc.VectorSubcoreMesh(core_axis_name="c", subcore_axis_name="s")


def _fill_buf(buf, value):
    rows, cols = buf.shape
    vec = jnp.full((16,), value, jnp.float32)

    def row(i, carry):
        for k in range(cols // 16):
            buf[i, pl.ds(k * 16, 16)] = vec
        return carry

    lax.fori_loop(0, rows, row, 0)


@functools.cache
def _make_deg_kernel():
    return pl.kernel(
        _deg_body,
        out_type=jax.ShapeDtypeStruct((NC, N_ACC, DW), jnp.float32),
        mesh=_mesh(),
        scratch_types=[
            pltpu.VMEM((SEG, CH), jnp.int32),
            pltpu.VMEM((CH, DW), jnp.float32),
            pltpu.VMEM((CH, DW), jnp.float32),
            pltpu.VMEM_SHARED((N_ACC, DW), jnp.float32),
        ],
    )


def _deg_body(dst_hbm, out_hbm, idx_v, ones_v, zb, acc):
    c = lax.axis_index("c")
    s = lax.axis_index("s")
    wid = s * NC + c

    _fill_buf(ones_v, 1.0)
    _fill_buf(zb, 0.0)
    base = s * RPT

    def zchunk(k, carry):
        pltpu.sync_copy(zb, acc.at[pl.ds(base + k * CH, CH)])
        return carry

    lax.fori_loop(0, RPT // CH, zchunk, 0)
    plsc.subcore_barrier()

    def seg_body(g, carry):
        pltpu.sync_copy(dst_hbm.at[wid, g], idx_v)

        def body(j, carry2):
            pltpu.sync_copy(ones_v, acc.at[idx_v.at[j]], add=True)
            return carry2

        lax.fori_loop(0, SEG, body, 0)
        return carry

    lax.fori_loop(0, NSEG, seg_body, 0)
    plsc.subcore_barrier()

    def wb(k, carry):
        pltpu.sync_copy(
            acc.at[pl.ds(base + k * CH, CH)],
            out_hbm.at[c, pl.ds(base + k * CH, CH)],
        )
        return carry

    lax.fori_loop(0, RPT // CH, wb, 0)


@functools.cache
def _make_scatter_kernel():
    return pl.kernel(
        _scatter_body,
        out_type=jax.ShapeDtypeStruct((NC, N_ACC, HH), jnp.float32),
        mesh=_mesh(),
        scratch_types=[
            pltpu.VMEM((SEG, CH), jnp.int32),
            pltpu.VMEM((SEG, CH), jnp.int32),
            pltpu.VMEM((CH, HH), jnp.float32),
            pltpu.VMEM((CH, HH), jnp.float32),
            pltpu.VMEM_SHARED((N_ACC, HH), jnp.float32),
            pltpu.SemaphoreType.DMA,
            pltpu.SemaphoreType.DMA,
        ],
    )


def _scatter_body(m_hbm, src_hbm, dst_hbm, out_hbm, si_v, di_v, buf_a,
                  buf_b, acc, sem_a, sem_b):
    c = lax.axis_index("c")
    s = lax.axis_index("s")
    wid = s * NC + c

    _fill_buf(buf_a, 0.0)
    base = s * RPT

    def zchunk(k, carry):
        pltpu.sync_copy(buf_a, acc.at[pl.ds(base + k * CH, CH)])
        return carry

    lax.fori_loop(0, RPT // CH, zchunk, 0)
    plsc.subcore_barrier()

    def seg_body(g, carry):
        pltpu.sync_copy(src_hbm.at[wid, g], si_v)
        pltpu.sync_copy(dst_hbm.at[wid, g], di_v)
        pltpu.async_copy(m_hbm.at[si_v.at[0]], buf_a, sem_a)

        def body(j2, carry2):
            a = 2 * j2
            b = a + 1
            pltpu.async_copy(m_hbm.at[si_v.at[b]], buf_b, sem_b)
            pltpu.make_async_copy(m_hbm.at[si_v.at[a]], buf_a, sem_a).wait()
            pltpu.sync_copy(buf_a, acc.at[di_v.at[a]], add=True)

            @pl.when(b + 1 < SEG)
            def _():
                pltpu.async_copy(m_hbm.at[si_v.at[b + 1]], buf_a, sem_a)

            pltpu.make_async_copy(m_hbm.at[si_v.at[b]], buf_b, sem_b).wait()
            pltpu.sync_copy(buf_b, acc.at[di_v.at[b]], add=True)
            return carry2

        lax.fori_loop(0, SEG // 2, body, 0)
        return carry

    lax.fori_loop(0, NSEG, seg_body, 0)
    plsc.subcore_barrier()

    def wb(k, carry):
        pltpu.sync_copy(
            acc.at[pl.ds(base + k * CH, CH)],
            out_hbm.at[c, pl.ds(base + k * CH, CH)],
        )
        return carry

    lax.fori_loop(0, RPT // CH, wb, 0)


def _k0_body(x_ref, we_ref, be_ref, w0_ref, p_ref, m_ref, dinv_ref):
    h = jnp.maximum(
        jnp.dot(x_ref[...], we_ref[...], preferred_element_type=jnp.float32)
        + be_ref[...], 0.0)
    deg = p_ref[0, :NN, :1] + p_ref[1, :NN, :1] + 1.0
    dinv = lax.rsqrt(deg)
    dinv_ref[...] = dinv
    m_ref[...] = dinv * jnp.dot(h, w0_ref[...],
                                preferred_element_type=jnp.float32)


def _kmid_body(s_ref, m_ref, dinv_ref, b_ref, w_ref, mo_ref):
    ssum = s_ref[0, :NN, :] + s_ref[1, :NN, :]
    dinv = dinv_ref[...]
    h = jnp.maximum(dinv * (ssum + m_ref[...]) + b_ref[...], 0.0)
    mo_ref[...] = dinv * jnp.dot(h, w_ref[...],
                                 preferred_element_type=jnp.float32)


def _kpost_body(s_ref, m_ref, dinv_ref, b_ref, batch_ref, wa_ref, ba_ref,
                wt_ref, bt_ref, wm_ref, bm_ref, h_ref, act_ref, tgt_ref,
                atom_ref):
    ssum = s_ref[0, :NN, :] + s_ref[1, :NN, :]
    dinv = dinv_ref[...]
    h = jnp.maximum(dinv * (ssum + m_ref[...]) + b_ref[...], 0.0)
    h_ref[...] = h
    seg = lax.broadcasted_iota(jnp.int32, (1, GG), 1)
    oh = (batch_ref[...] == seg).astype(jnp.float32)
    gsum = lax.dot_general(oh, h, (((0,), (0,)), ((), ())),
                           preferred_element_type=jnp.float32)
    counts = lax.dot_general(oh, jnp.ones((NN, 1), jnp.float32),
                             (((0,), (0,)), ((), ())),
                             preferred_element_type=jnp.float32)
    ge = gsum / jnp.maximum(counts, 1.0)
    act_ref[...] = jnp.dot(ge, wa_ref[...],
                           preferred_element_type=jnp.float32) + ba_ref[...]
    atom_ref[...] = jnp.dot(ge, wm_ref[...],
                            preferred_element_type=jnp.float32) + bm_ref[...]
    tgt_ref[...] = jnp.dot(h, wt_ref[...],
                           preferred_element_type=jnp.float32) + bt_ref[...]


_k0 = pl.pallas_call(
    _k0_body,
    out_shape=(
        jax.ShapeDtypeStruct((NN, HH), jnp.float32),
        jax.ShapeDtypeStruct((NN, 1), jnp.float32),
    ),
)

_kmid = pl.pallas_call(
    _kmid_body,
    out_shape=jax.ShapeDtypeStruct((NN, HH), jnp.float32),
)

_kpost = pl.pallas_call(
    _kpost_body,
    out_shape=(
        jax.ShapeDtypeStruct((NN, HH), jnp.float32),
        jax.ShapeDtypeStruct((GG, 4), jnp.float32),
        jax.ShapeDtypeStruct((NN, 1), jnp.float32),
        jax.ShapeDtypeStruct((GG, 16), jnp.float32),
    ),
)


def kernel(x, edge_index, batch, W_emb, b_emb, W0, b0, W1, b1, W2, b2,
           W_act, b_act, W_tgt, b_tgt, W_atom, b_atom):
    pad = E_PAD - EE
    srcp = jnp.concatenate(
        [edge_index[0], jnp.zeros((pad,), jnp.int32)]).reshape(
            NW, NSEG, SEG, CH)
    dstp = jnp.concatenate(
        [edge_index[1], jnp.full((pad,), NN, jnp.int32)]).reshape(
            NW, NSEG, SEG, CH)

    deg_k = _make_deg_kernel()
    scat_k = _make_scatter_kernel()
    degp = deg_k(dstp)
    m0, dinv = _k0(x, W_emb, b_emb.reshape(1, HH), W0, degp)
    s0 = scat_k(m0, srcp, dstp)
    m1 = _kmid(s0, m0, dinv, b0.reshape(1, HH), W1)
    s1 = scat_k(m1, srcp, dstp)
    m2 = _kmid(s1, m1, dinv, b1.reshape(1, HH), W2)
    s2 = scat_k(m2, srcp, dstp)
    h, act, tgt, atom = _kpost(
        s2, m2, dinv, b2.reshape(1, HH), batch.reshape(NN, 1),
        W_act, b_act.reshape(1, 4), W_tgt, b_tgt.reshape(1, 1),
        W_atom, b_atom.reshape(1, 16))
    return act, tgt[:, 0], atom, h


# speedup vs baseline: 8.5235x; 1.1597x over previous
import functools

import jax
import jax.numpy as jnp
from jax import lax
from jax.experimental import pallas as pl
from jax.experimental.pallas import tpu as pltpu
from jax.experimental.pallas import tpu_sc as plsc

NN = 10000
EE = 320000
HH = 128
GG = 64
NC = 2
NS = 16
NW = NC * NS
CH = 128
CPT = 80
SEG = 16
NSEG = CPT // SEG
EPT = CPT * CH
E_PAD = NW * EPT
N_ACC = 10240
RPT = N_ACC // NS
DW = 16

_mesh = lambda: plsc.VectorSubcoreMesh(core_axis_name="c", subcore_axis_name="s")


def _fill_buf(buf, value):
    rows, cols = buf.shape
    vec = jnp.full((16,), value, jnp.float32)

    def row(i, carry):
        for k in range(cols // 16):
            buf[i, pl.ds(k * 16, 16)] = vec
        return carry

    lax.fori_loop(0, rows, row, 0)


@functools.cache
def _make_deg_kernel():
    return pl.kernel(
        _deg_body,
        out_type=jax.ShapeDtypeStruct((NC, N_ACC, DW), jnp.float32),
        mesh=_mesh(),
        scratch_types=[
            pltpu.VMEM((SEG, CH), jnp.int32),
            pltpu.VMEM((CH, DW), jnp.float32),
            pltpu.VMEM((CH, DW), jnp.float32),
            pltpu.VMEM_SHARED((N_ACC, DW), jnp.float32),
        ],
    )


def _deg_body(dst_hbm, out_hbm, idx_v, ones_v, zb, acc):
    c = lax.axis_index("c")
    s = lax.axis_index("s")
    wid = s * NC + c

    _fill_buf(ones_v, 1.0)
    _fill_buf(zb, 0.0)
    base = s * RPT

    def zchunk(k, carry):
        pltpu.sync_copy(zb, acc.at[pl.ds(base + k * CH, CH)])
        return carry

    lax.fori_loop(0, RPT // CH, zchunk, 0)
    plsc.subcore_barrier()

    def seg_body(g, carry):
        pltpu.sync_copy(dst_hbm.at[wid, g], idx_v)

        def body(j, carry2):
            pltpu.sync_copy(ones_v, acc.at[idx_v.at[j]], add=True)
            return carry2

        lax.fori_loop(0, SEG, body, 0)
        return carry

    lax.fori_loop(0, NSEG, seg_body, 0)
    plsc.subcore_barrier()

    def wb(k, carry):
        pltpu.sync_copy(
            acc.at[pl.ds(base + k * CH, CH)],
            out_hbm.at[c, pl.ds(base + k * CH, CH)],
        )
        return carry

    lax.fori_loop(0, RPT // CH, wb, 0)


@functools.cache
def _make_scatter_kernel():
    return pl.kernel(
        _scatter_body,
        out_type=jax.ShapeDtypeStruct((NC, N_ACC, HH), jnp.float32),
        mesh=_mesh(),
        scratch_types=[
            pltpu.VMEM((SEG, CH), jnp.int32),
            pltpu.VMEM((SEG, CH), jnp.int32),
            pltpu.VMEM((CH, HH), jnp.float32),
            pltpu.VMEM((CH, HH), jnp.float32),
            pltpu.VMEM_SHARED((N_ACC, HH), jnp.float32),
            pltpu.SemaphoreType.DMA,
            pltpu.SemaphoreType.DMA,
        ],
    )


def _scatter_body(m_hbm, src_hbm, dst_hbm, out_hbm, si_v, di_v, buf_a,
                  buf_b, acc, sem_a, sem_b):
    c = lax.axis_index("c")
    s = lax.axis_index("s")
    wid = s * NC + c

    _fill_buf(buf_a, 0.0)
    base = s * RPT

    def zchunk(k, carry):
        pltpu.sync_copy(buf_a, acc.at[pl.ds(base + k * CH, CH)])
        return carry

    lax.fori_loop(0, RPT // CH, zchunk, 0)
    plsc.subcore_barrier()

    def seg_body(g, carry):
        pltpu.sync_copy(src_hbm.at[wid, g], si_v)
        pltpu.sync_copy(dst_hbm.at[wid, g], di_v)
        pltpu.async_copy(m_hbm.at[si_v.at[0]], buf_a, sem_a)

        def body(j2, carry2):
            a = 2 * j2
            b = a + 1
            pltpu.async_copy(m_hbm.at[si_v.at[b]], buf_b, sem_b)
            pltpu.make_async_copy(m_hbm.at[si_v.at[a]], buf_a, sem_a).wait()
            pltpu.sync_copy(buf_a, acc.at[di_v.at[a]], add=True)

            @pl.when(b + 1 < SEG)
            def _():
                pltpu.async_copy(m_hbm.at[si_v.at[b + 1]], buf_a, sem_a)

            pltpu.make_async_copy(m_hbm.at[si_v.at[b]], buf_b, sem_b).wait()
            pltpu.sync_copy(buf_b, acc.at[di_v.at[b]], add=True)
            return carry2

        lax.fori_loop(0, SEG // 2, body, 0)
        return carry

    lax.fori_loop(0, NSEG, seg_body, 0)
    plsc.subcore_barrier()

    def wb(k, carry):
        pltpu.sync_copy(
            acc.at[pl.ds(base + k * CH, CH)],
            out_hbm.at[c, pl.ds(base + k * CH, CH)],
        )
        return carry

    lax.fori_loop(0, RPT // CH, wb, 0)


def _k0_body(x_ref, we_ref, be_ref, w0_ref, p_ref, m_ref, dinv_ref):
    h = jnp.maximum(
        jnp.dot(x_ref[...], we_ref[...], preferred_element_type=jnp.float32)
        + be_ref[...], 0.0)
    deg = p_ref[0, :NN, :1] + p_ref[1, :NN, :1] + 1.0
    dinv = lax.rsqrt(deg)
    dinv_ref[...] = dinv
    m_ref[...] = dinv * jnp.dot(h, w0_ref[...],
                                preferred_element_type=jnp.float32)


def _kmid_body(s_ref, m_ref, dinv_ref, b_ref, w_ref, mo_ref):
    ssum = s_ref[0, :NN, :] + s_ref[1, :NN, :]
    dinv = dinv_ref[...]
    h = jnp.maximum(dinv * (ssum + m_ref[...]) + b_ref[...], 0.0)
    mo_ref[...] = dinv * jnp.dot(h, w_ref[...],
                                 preferred_element_type=jnp.float32)


def _kpost_body(s_ref, m_ref, dinv_ref, b_ref, batch_ref, wa_ref, ba_ref,
                wt_ref, bt_ref, wm_ref, bm_ref, h_ref, act_ref, tgt_ref,
                atom_ref):
    ssum = s_ref[0, :NN, :] + s_ref[1, :NN, :]
    dinv = dinv_ref[...]
    h = jnp.maximum(dinv * (ssum + m_ref[...]) + b_ref[...], 0.0)
    h_ref[...] = h
    seg = lax.broadcasted_iota(jnp.int32, (1, GG), 1)
    oh = (batch_ref[...] == seg).astype(jnp.float32)
    gsum = lax.dot_general(oh, h, (((0,), (0,)), ((), ())),
                           preferred_element_type=jnp.float32)
    counts = lax.dot_general(oh, jnp.ones((NN, 1), jnp.float32),
                             (((0,), (0,)), ((), ())),
                             preferred_element_type=jnp.float32)
    ge = gsum / jnp.maximum(counts, 1.0)
    act_ref[...] = jnp.dot(ge, wa_ref[...],
                           preferred_element_type=jnp.float32) + ba_ref[...]
    atom_ref[...] = jnp.dot(ge, wm_ref[...],
                            preferred_element_type=jnp.float32) + bm_ref[...]
    tgt_ref[...] = jnp.dot(h, wt_ref[...],
                           preferred_element_type=jnp.float32) + bt_ref[...]


_k0 = pl.pallas_call(
    _k0_body,
    out_shape=(
        jax.ShapeDtypeStruct((NN, HH), jnp.float32),
        jax.ShapeDtypeStruct((NN, 1), jnp.float32),
    ),
)

_kmid = pl.pallas_call(
    _kmid_body,
    out_shape=jax.ShapeDtypeStruct((NN, HH), jnp.float32),
)

_kpost = pl.pallas_call(
    _kpost_body,
    out_shape=(
        jax.ShapeDtypeStruct((NN, HH), jnp.float32),
        jax.ShapeDtypeStruct((GG, 4), jnp.float32),
        jax.ShapeDtypeStruct((NN, 1), jnp.float32),
        jax.ShapeDtypeStruct((GG, 16), jnp.float32),
    ),
)


def kernel(x, edge_index, batch, W_emb, b_emb, W0, b0, W1, b1, W2, b2,
           W_act, b_act, W_tgt, b_tgt, W_atom, b_atom):
    pad = E_PAD - EE
    srcp = jnp.concatenate(
        [edge_index[0], jnp.zeros((pad,), jnp.int32)]).reshape(
            NW, NSEG, SEG, CH)
    dstp = jnp.concatenate(
        [edge_index[1], jnp.full((pad,), NN, jnp.int32)]).reshape(
            NW, NSEG, SEG, CH)

    deg_k = _make_deg_kernel()
    scat_k = _make_scatter_kernel()
    degp = deg_k(dstp)
    m0, dinv = _k0(x, W_emb, b_emb.reshape(1, HH), W0, degp)
    s0 = scat_k(m0, srcp, dstp)
    m1 = _kmid(s0, m0, dinv, b0.reshape(1, HH), W1)
    s1 = scat_k(m1, srcp, dstp)
    m2 = _kmid(s1, m1, dinv, b1.reshape(1, HH), W2)
    s2 = scat_k(m2, srcp, dstp)
    h, act, tgt, atom = _kpost(
        s2, m2, dinv, b2.reshape(1, HH), batch.reshape(NN, 1),
        W_act, b_act.reshape(1, 4), W_tgt, b_tgt.reshape(1, 1),
        W_atom, b_atom.reshape(1, 16))
    return act, tgt[:, 0], atom, h


# speedup vs baseline: 9.4752x; 1.1117x over previous
import functools

import jax
import jax.numpy as jnp
from jax import lax
from jax.experimental import pallas as pl
from jax.experimental.pallas import tpu as pltpu
from jax.experimental.pallas import tpu_sc as plsc

NN = 10000
EE = 320000
HH = 128
GG = 64
NC = 2
NS = 16
NW = NC * NS
CH = 128
SEG = 8
SPP = 20
Q0 = 15
TOT_SEG = NS * SPP
E_PAD = TOT_SEG * SEG * CH
N_ACC = 10240
RPT = N_ACC // NS
DW = 16

_mesh = lambda: plsc.VectorSubcoreMesh(core_axis_name="c", subcore_axis_name="s")


def _fill_buf(buf, value):
    rows, cols = buf.shape
    vec = jnp.full((16,), value, jnp.float32)

    def row(i, carry):
        for k in range(cols // 16):
            buf[i, pl.ds(k * 16, 16)] = vec
        return carry

    lax.fori_loop(0, rows, row, 0)


@functools.cache
def _make_deg_kernel():
    return pl.kernel(
        _deg_body,
        out_type=jax.ShapeDtypeStruct((NC, N_ACC, DW), jnp.float32),
        mesh=_mesh(),
        scratch_types=[
            pltpu.VMEM((SEG, CH), jnp.int32),
            pltpu.VMEM((CH, DW), jnp.float32),
            pltpu.VMEM((CH, DW), jnp.float32),
            pltpu.VMEM_SHARED((N_ACC, DW), jnp.float32),
        ],
    )


def _deg_body(dst_hbm, out_hbm, idx_v, ones_v, zb, acc):
    c = lax.axis_index("c")
    s = lax.axis_index("s")
    sb = s * SPP + c * (SPP // 2)

    _fill_buf(ones_v, 1.0)
    _fill_buf(zb, 0.0)
    base = s * RPT

    def zchunk(k, carry):
        pltpu.sync_copy(zb, acc.at[pl.ds(base + k * CH, CH)])
        return carry

    lax.fori_loop(0, RPT // CH, zchunk, 0)
    plsc.subcore_barrier()

    def seg_body(g, carry):
        pltpu.sync_copy(dst_hbm.at[sb + g], idx_v)

        def body(j, carry2):
            pltpu.sync_copy(ones_v, acc.at[idx_v.at[j]], add=True)
            return carry2

        lax.fori_loop(0, SEG, body, 0)
        return carry

    lax.fori_loop(0, SPP // 2, seg_body, 0)
    plsc.subcore_barrier()

    def wb(k, carry):
        pltpu.sync_copy(
            acc.at[pl.ds(base + k * CH, CH)],
            out_hbm.at[c, pl.ds(base + k * CH, CH)],
        )
        return carry

    lax.fori_loop(0, RPT // CH, wb, 0)


@functools.cache
def _make_scatter_kernel():
    return pl.kernel(
        _scatter_body,
        out_type=jax.ShapeDtypeStruct((NC, N_ACC, HH), jnp.float32),
        mesh=_mesh(),
        scratch_types=[
            pltpu.VMEM((SEG, CH), jnp.int32),
            pltpu.VMEM((SEG, CH), jnp.int32),
            pltpu.VMEM((CH, HH), jnp.float32),
            pltpu.VMEM((CH, HH), jnp.float32),
            pltpu.VMEM_SHARED((N_ACC, HH), jnp.float32),
            pltpu.SemaphoreType.DMA,
            pltpu.SemaphoreType.DMA,
        ],
    )


def _scatter_body(m_hbm, src_hbm, dst_hbm, out_hbm, si_v, di_v, buf_a,
                  buf_b, acc, sem_a, sem_b):
    c = lax.axis_index("c")
    s = lax.axis_index("s")
    nseg = jnp.where(c == 0, Q0, SPP - Q0)
    sb = s * SPP + c * Q0

    _fill_buf(buf_a, 0.0)
    base = s * RPT

    def zchunk(k, carry):
        pltpu.sync_copy(buf_a, acc.at[pl.ds(base + k * CH, CH)])
        return carry

    lax.fori_loop(0, RPT // CH, zchunk, 0)
    plsc.subcore_barrier()

    def seg_body(g, carry):
        pltpu.sync_copy(src_hbm.at[sb + g], si_v)
        pltpu.sync_copy(dst_hbm.at[sb + g], di_v)
        pltpu.async_copy(m_hbm.at[si_v.at[0]], buf_a, sem_a)

        def body(j2, carry2):
            a = 2 * j2
            b = a + 1
            pltpu.async_copy(m_hbm.at[si_v.at[b]], buf_b, sem_b)
            pltpu.make_async_copy(m_hbm.at[si_v.at[a]], buf_a, sem_a).wait()
            pltpu.sync_copy(buf_a, acc.at[di_v.at[a]], add=True)

            @pl.when(b + 1 < SEG)
            def _():
                pltpu.async_copy(m_hbm.at[si_v.at[b + 1]], buf_a, sem_a)

            pltpu.make_async_copy(m_hbm.at[si_v.at[b]], buf_b, sem_b).wait()
            pltpu.sync_copy(buf_b, acc.at[di_v.at[b]], add=True)
            return carry2

        lax.fori_loop(0, SEG // 2, body, 0)
        return carry

    lax.fori_loop(0, nseg, seg_body, 0)
    plsc.subcore_barrier()

    def wb(k, carry):
        pltpu.sync_copy(
            acc.at[pl.ds(base + k * CH, CH)],
            out_hbm.at[c, pl.ds(base + k * CH, CH)],
        )
        return carry

    lax.fori_loop(0, RPT // CH, wb, 0)


def _k0_body(x_ref, we_ref, be_ref, w0_ref, p_ref, m_ref, dinv_ref):
    h = jnp.maximum(
        jnp.dot(x_ref[...], we_ref[...], preferred_element_type=jnp.float32)
        + be_ref[...], 0.0)
    deg = p_ref[0, :NN, :1] + p_ref[1, :NN, :1] + 1.0
    dinv = lax.rsqrt(deg)
    dinv_ref[...] = dinv
    m_ref[...] = dinv * jnp.dot(h, w0_ref[...],
                                preferred_element_type=jnp.float32)


def _kmid_body(s_ref, m_ref, dinv_ref, b_ref, w_ref, mo_ref):
    ssum = s_ref[0, :NN, :] + s_ref[1, :NN, :]
    dinv = dinv_ref[...]
    h = jnp.maximum(dinv * (ssum + m_ref[...]) + b_ref[...], 0.0)
    mo_ref[...] = dinv * jnp.dot(h, w_ref[...],
                                 preferred_element_type=jnp.float32)


def _kpost_body(s_ref, m_ref, dinv_ref, b_ref, batch_ref, wa_ref, ba_ref,
                wt_ref, bt_ref, wm_ref, bm_ref, h_ref, act_ref, tgt_ref,
                atom_ref):
    ssum = s_ref[0, :NN, :] + s_ref[1, :NN, :]
    dinv = dinv_ref[...]
    h = jnp.maximum(dinv * (ssum + m_ref[...]) + b_ref[...], 0.0)
    h_ref[...] = h
    seg = lax.broadcasted_iota(jnp.int32, (1, GG), 1)
    oh = (batch_ref[...] == seg).astype(jnp.float32)
    gsum = lax.dot_general(oh, h, (((0,), (0,)), ((), ())),
                           preferred_element_type=jnp.float32)
    counts = lax.dot_general(oh, jnp.ones((NN, 1), jnp.float32),
                             (((0,), (0,)), ((), ())),
                             preferred_element_type=jnp.float32)
    ge = gsum / jnp.maximum(counts, 1.0)
    act_ref[...] = jnp.dot(ge, wa_ref[...],
                           preferred_element_type=jnp.float32) + ba_ref[...]
    atom_ref[...] = jnp.dot(ge, wm_ref[...],
                            preferred_element_type=jnp.float32) + bm_ref[...]
    tgt_ref[...] = jnp.dot(h, wt_ref[...],
                           preferred_element_type=jnp.float32) + bt_ref[...]


_k0 = pl.pallas_call(
    _k0_body,
    out_shape=(
        jax.ShapeDtypeStruct((NN, HH), jnp.float32),
        jax.ShapeDtypeStruct((NN, 1), jnp.float32),
    ),
)

_kmid = pl.pallas_call(
    _kmid_body,
    out_shape=jax.ShapeDtypeStruct((NN, HH), jnp.float32),
)

_kpost = pl.pallas_call(
    _kpost_body,
    out_shape=(
        jax.ShapeDtypeStruct((NN, HH), jnp.float32),
        jax.ShapeDtypeStruct((GG, 4), jnp.float32),
        jax.ShapeDtypeStruct((NN, 1), jnp.float32),
        jax.ShapeDtypeStruct((GG, 16), jnp.float32),
    ),
)


def kernel(x, edge_index, batch, W_emb, b_emb, W0, b0, W1, b1, W2, b2,
           W_act, b_act, W_tgt, b_tgt, W_atom, b_atom):
    pad = E_PAD - EE
    srcp = jnp.concatenate(
        [edge_index[0], jnp.zeros((pad,), jnp.int32)]).reshape(
            TOT_SEG, SEG, CH)
    dstp = jnp.concatenate(
        [edge_index[1], jnp.full((pad,), NN, jnp.int32)]).reshape(
            TOT_SEG, SEG, CH)

    deg_k = _make_deg_kernel()
    scat_k = _make_scatter_kernel()
    degp = deg_k(dstp)
    m0, dinv = _k0(x, W_emb, b_emb.reshape(1, HH), W0, degp)
    s0 = scat_k(m0, srcp, dstp)
    m1 = _kmid(s0, m0, dinv, b0.reshape(1, HH), W1)
    s1 = scat_k(m1, srcp, dstp)
    m2 = _kmid(s1, m1, dinv, b1.reshape(1, HH), W2)
    s2 = scat_k(m2, srcp, dstp)
    h, act, tgt, atom = _kpost(
        s2, m2, dinv, b2.reshape(1, HH), batch.reshape(NN, 1),
        W_act, b_act.reshape(1, 4), W_tgt, b_tgt.reshape(1, 1),
        W_atom, b_atom.reshape(1, 16))
    return act, tgt[:, 0], atom, h
